# Initial kernel scaffold; baseline (speedup 1.0000x reference)
import jax
import jax.numpy as jnp
from jax.experimental import pallas as pl


def kernel(inputs, table, W1, b1, W2, b2):
    raise NotImplementedError("write your pallas kernel here")


# speedup vs baseline: 1.9680x; 1.9680x over previous
import functools

import jax
import jax.numpy as jnp
from jax import lax
from jax.experimental import pallas as pl
from jax.experimental.pallas import tpu as pltpu
from jax.experimental.pallas import tpu_sc as plsc

VOCAB = 1000000
EMB = 64
HIDDEN = 256
OUT = 6
B = 16384
L = 200

LANES = 16
NVREG = EMB // LANES
SEG = 100
SEGS_PER_ROW = L // SEG
NC, NS = 2, 16
NW = NC * NS
BPW = B // NW
CH = 8
NSEG = CH * SEGS_PER_ROW
NCHUNK = BPW // CH

_mesh = plsc.VectorSubcoreMesh(core_axis_name="c", subcore_axis_name="s")


@functools.partial(
    pl.kernel,
    out_type=jax.ShapeDtypeStruct((B, EMB), jnp.float32),
    mesh=_mesh,
    scratch_types=[
        pltpu.VMEM((NSEG, SEG), jnp.int32),
        pltpu.VMEM((SEG, EMB), jnp.float32),
        pltpu.VMEM((SEG, EMB), jnp.float32),
        pltpu.VMEM((CH, EMB), jnp.float32),
        pltpu.SemaphoreType.DMA,
        pltpu.SemaphoreType.DMA,
    ],
    compiler_params=pltpu.CompilerParams(use_tc_tiling_on_sc=False),
)
def _pool(idx_hbm, table_hbm, out_hbm, idx_v, rows0, rows1, out_v, sem0, sem1):
    wid = lax.axis_index("s") * NC + lax.axis_index("c")
    base = wid * BPW
    bufs = (rows0, rows1)
    sems = (sem0, sem1)
    inv_l = 1.0 / L

    def chunk(c, carry):
        row0 = base + c * CH
        pltpu.sync_copy(idx_hbm.at[pl.ds(row0 * SEGS_PER_ROW, NSEG)], idx_v)
        pltpu.async_copy(table_hbm.at[idx_v.at[0]], bufs[0], sems[0])
        for e in range(CH):
            accs = [jnp.zeros((LANES,), jnp.float32) for _ in range(NVREG)]
            for s2 in range(SEGS_PER_ROW):
                s = e * SEGS_PER_ROW + s2
                buf = bufs[s % 2]
                pltpu.make_async_copy(table_hbm.at[idx_v.at[s]], buf, sems[s % 2]).wait()
                if s + 1 < NSEG:
                    pltpu.async_copy(
                        table_hbm.at[idx_v.at[s + 1]], bufs[(s + 1) % 2], sems[(s + 1) % 2]
                    )

                def jbody(j, a, _buf=buf):
                    return [a[k] + _buf[j, pl.ds(LANES * k, LANES)] for k in range(NVREG)]

                accs = lax.fori_loop(0, SEG, jbody, accs)
            for k in range(NVREG):
                out_v[e, pl.ds(LANES * k, LANES)] = accs[k] * inv_l
        pltpu.sync_copy(out_v, out_hbm.at[pl.ds(row0, CH)])
        return carry

    lax.fori_loop(0, NCHUNK, chunk, 0)


OUTP = 128
BM = 2048


def _mlp_body(x_ref, w1_ref, b1_ref, w2_ref, b2_ref, o_ref):
    h = jnp.dot(x_ref[...], w1_ref[...], preferred_element_type=jnp.float32)
    h = jnp.maximum(h + b1_ref[...], 0.0)
    z = jnp.dot(h, w2_ref[...], preferred_element_type=jnp.float32) + b2_ref[...]
    o_ref[...] = 1.0 / (1.0 + jnp.exp(-z))


def _mlp(x, W1, b1, W2, b2):
    w2p = jnp.zeros((HIDDEN, OUTP), jnp.float32).at[:, :OUT].set(W2)
    b2p = jnp.zeros((1, OUTP), jnp.float32).at[:, :OUT].set(b2)
    out = pl.pallas_call(
        _mlp_body,
        grid=(B // BM,),
        in_specs=[
            pl.BlockSpec((BM, EMB), lambda i: (i, 0)),
            pl.BlockSpec((EMB, HIDDEN), lambda i: (0, 0)),
            pl.BlockSpec((1, HIDDEN), lambda i: (0, 0)),
            pl.BlockSpec((HIDDEN, OUTP), lambda i: (0, 0)),
            pl.BlockSpec((1, OUTP), lambda i: (0, 0)),
        ],
        out_specs=pl.BlockSpec((BM, OUTP), lambda i: (i, 0)),
        out_shape=jax.ShapeDtypeStruct((B, OUTP), jnp.float32),
    )(x, W1, b1.reshape(1, HIDDEN), w2p, b2p)
    return out[:, :OUT]


def kernel(inputs, table, W1, b1, W2, b2):
    idx = inputs.astype(jnp.int32).reshape(B * SEGS_PER_ROW, SEG)
    pooled = _pool(idx, table)
    return _mlp(pooled, W1, b1, W2, b2)


# speedup vs baseline: 2.5469x; 1.2941x over previous
import functools

import jax
import jax.numpy as jnp
from jax import lax
from jax.experimental import pallas as pl
from jax.experimental.pallas import tpu as pltpu
from jax.experimental.pallas import tpu_sc as plsc

VOCAB = 1000000
EMB = 64
HIDDEN = 256
OUT = 6
B = 16384
L = 200

LANES = 16
NVREG = EMB // LANES
SEG = 100
SEGS_PER_ROW = L // SEG
NC, NS = 2, 16
NW = NC * NS
BPW = B // NW
NSEGW = BPW * SEGS_PER_ROW
HALF = NSEGW // 2
ROWS_H = HALF // SEGS_PER_ROW

_mesh = plsc.VectorSubcoreMesh(core_axis_name="c", subcore_axis_name="s")


@functools.partial(
    pl.kernel,
    out_type=jax.ShapeDtypeStruct((B, EMB), jnp.float32),
    mesh=_mesh,
    scratch_types=[
        pltpu.VMEM((HALF, SEG), jnp.int32),
        pltpu.VMEM((SEG, EMB), jnp.float32),
        pltpu.VMEM((SEG, EMB), jnp.float32),
        pltpu.VMEM((BPW, EMB), jnp.float32),
        pltpu.SemaphoreType.DMA,
        pltpu.SemaphoreType.DMA,
    ],
    compiler_params=pltpu.CompilerParams(use_tc_tiling_on_sc=False),
)
def _pool(idx_hbm, table_hbm, out_hbm, idx_v, rows0, rows1, out_v, sem0, sem1):
    wid = lax.axis_index("s") * NC + lax.axis_index("c")
    base = wid * BPW
    bufs = (rows0, rows1)
    sems = (sem0, sem1)
    inv_l = 1.0 / L

    def seg_acc(buf, accs):
        def jbody(j, a):
            return [a[k] + buf[j, pl.ds(LANES * k, LANES)] for k in range(NVREG)]

        return lax.fori_loop(0, SEG, jbody, accs, unroll=5)

    for h in range(2):
        pltpu.sync_copy(idx_hbm.at[pl.ds((base + h * ROWS_H) * SEGS_PER_ROW, HALF)], idx_v)
        pltpu.async_copy(table_hbm.at[idx_v.at[0]], bufs[0], sems[0])
        pltpu.async_copy(table_hbm.at[idx_v.at[1]], bufs[1], sems[1])

        def row_body(e, carry, _h=h):
            accs = [jnp.zeros((LANES,), jnp.float32) for _ in range(NVREG)]
            for b in range(2):
                s = 2 * e + b
                pltpu.make_async_copy(table_hbm.at[idx_v.at[s]], bufs[b], sems[b]).wait()
                accs = seg_acc(bufs[b], accs)
                pltpu.async_copy(table_hbm.at[idx_v.at[s + 2]], bufs[b], sems[b])
            r = _h * ROWS_H + e
            for k in range(NVREG):
                out_v[r, pl.ds(LANES * k, LANES)] = accs[k] * inv_l
            return carry

        lax.fori_loop(0, ROWS_H - 1, row_body, 0)

        accs = [jnp.zeros((LANES,), jnp.float32) for _ in range(NVREG)]
        for b in range(2):
            s = 2 * (ROWS_H - 1) + b
            pltpu.make_async_copy(table_hbm.at[idx_v.at[s]], bufs[b], sems[b]).wait()
            accs = seg_acc(bufs[b], accs)
        r = h * ROWS_H + (ROWS_H - 1)
        for k in range(NVREG):
            out_v[r, pl.ds(LANES * k, LANES)] = accs[k] * inv_l

    pltpu.sync_copy(out_v, out_hbm.at[pl.ds(base, BPW)])


OUTP = 128
BM = 2048


def _mlp_body(x_ref, w1_ref, b1_ref, w2_ref, b2_ref, o_ref):
    h = jnp.dot(x_ref[...], w1_ref[...], preferred_element_type=jnp.float32)
    h = jnp.maximum(h + b1_ref[...], 0.0)
    z = jnp.dot(h, w2_ref[...], preferred_element_type=jnp.float32) + b2_ref[...]
    o_ref[...] = 1.0 / (1.0 + jnp.exp(-z))


def _mlp(x, W1, b1, W2, b2):
    w2p = jnp.zeros((HIDDEN, OUTP), jnp.float32).at[:, :OUT].set(W2)
    b2p = jnp.zeros((1, OUTP), jnp.float32).at[:, :OUT].set(b2)
    out = pl.pallas_call(
        _mlp_body,
        grid=(B // BM,),
        in_specs=[
            pl.BlockSpec((BM, EMB), lambda i: (i, 0)),
            pl.BlockSpec((EMB, HIDDEN), lambda i: (0, 0)),
            pl.BlockSpec((1, HIDDEN), lambda i: (0, 0)),
            pl.BlockSpec((HIDDEN, OUTP), lambda i: (0, 0)),
            pl.BlockSpec((1, OUTP), lambda i: (0, 0)),
        ],
        out_specs=pl.BlockSpec((BM, OUTP), lambda i: (i, 0)),
        out_shape=jax.ShapeDtypeStruct((B, OUTP), jnp.float32),
    )(x, W1, b1.reshape(1, HIDDEN), w2p, b2p)
    return out[:, :OUT]


def kernel(inputs, table, W1, b1, W2, b2):
    idx = inputs.astype(jnp.int32).reshape(B * SEGS_PER_ROW, SEG)
    pooled = _pool(idx, table)
    return _mlp(pooled, W1, b1, W2, b2)


# speedup vs baseline: 3.1890x; 1.2521x over previous
import functools

import jax
import jax.numpy as jnp
from jax import lax
from jax.experimental import pallas as pl
from jax.experimental.pallas import tpu as pltpu
from jax.experimental.pallas import tpu_sc as plsc

VOCAB = 1000000
EMB = 64
HIDDEN = 256
OUT = 6
B = 16384
L = 200

LANES = 16
NVREG = EMB // LANES
SEGA = 96
SEGB = L - SEGA
NC, NS = 2, 16
NW = NC * NS
BPW = B // NW
ROWS_H = BPW // 2
HALFV = VOCAB // 2

_mesh = plsc.VectorSubcoreMesh(core_axis_name="c", subcore_axis_name="s")


@functools.partial(
    pl.kernel,
    out_type=jax.ShapeDtypeStruct((B, EMB), jnp.float32),
    mesh=_mesh,
    scratch_types=[
        pltpu.VMEM((ROWS_H, L), jnp.int32),
        pltpu.VMEM((SEGA, EMB), jnp.float32),
        pltpu.VMEM((SEGB, EMB), jnp.float32),
        pltpu.VMEM((BPW, EMB), jnp.float32),
        pltpu.SemaphoreType.DMA,
        pltpu.SemaphoreType.DMA,
    ],
    compiler_params=pltpu.CompilerParams(use_tc_tiling_on_sc=False),
)
def _pool(idx_hbm, table_hbm, out_hbm, idx_v, rows0, rows1, out_v, sem0, sem1):
    wid = lax.axis_index("s") * NC + lax.axis_index("c")
    base = wid * BPW
    bufs = (rows0, rows1)
    sems = (sem0, sem1)
    seglen = (SEGA, SEGB)
    inv_l = 1.0 / L

    def seg(e, b):
        return idx_v.at[e, pl.ds(b * SEGA, seglen[b])]

    def seg_acc(buf, n, accs):
        def jbody(j, a):
            return [a[k] + buf[j, pl.ds(LANES * k, LANES)] for k in range(NVREG)]

        return lax.fori_loop(0, n, jbody, accs, unroll=4)

    for h in range(2):
        r0 = base + h * ROWS_H
        pltpu.sync_copy(idx_hbm.at[pl.ds(r0, ROWS_H)], idx_v)
        pltpu.async_copy(table_hbm.at[seg(0, 0)], bufs[0], sems[0])
        pltpu.async_copy(table_hbm.at[seg(0, 1)], bufs[1], sems[1])

        def row_body(e, carry, _h=h):
            accs = [jnp.zeros((LANES,), jnp.float32) for _ in range(NVREG)]
            for b in range(2):
                pltpu.make_async_copy(table_hbm.at[seg(e, b)], bufs[b], sems[b]).wait()
                accs = seg_acc(bufs[b], seglen[b], accs)
                pltpu.async_copy(table_hbm.at[seg(e + 1, b)], bufs[b], sems[b])
            r = _h * ROWS_H + e
            for k in range(NVREG):
                out_v[r, pl.ds(LANES * k, LANES)] = accs[k] * inv_l
            return carry

        lax.fori_loop(0, ROWS_H - 1, row_body, 0)

        accs = [jnp.zeros((LANES,), jnp.float32) for _ in range(NVREG)]
        for b in range(2):
            pltpu.make_async_copy(table_hbm.at[seg(ROWS_H - 1, b)], bufs[b], sems[b]).wait()
            accs = seg_acc(bufs[b], seglen[b], accs)
        r = h * ROWS_H + (ROWS_H - 1)
        for k in range(NVREG):
            out_v[r, pl.ds(LANES * k, LANES)] = accs[k] * inv_l

    pltpu.sync_copy(out_v, out_hbm.at[pl.ds(base, BPW)])


BHALF = 2048
NWIN = pl.cdiv(VOCAB, 2 * BHALF)
NPAIR = NWIN * BHALF
NBLK = pl.cdiv(VOCAB, BHALF)


def _repack_body(x1_ref, x2_ref, o_ref):
    o_ref[...] = jnp.concatenate([x1_ref[...].T, x2_ref[...].T], axis=1)


def _repack(table):
    tt = table.T
    return pl.pallas_call(
        _repack_body,
        grid=(NWIN,),
        in_specs=[
            pl.BlockSpec((EMB, BHALF), lambda i: (0, 2 * i)),
            pl.BlockSpec((EMB, BHALF), lambda i: (0, jnp.minimum(2 * i + 1, NBLK - 1))),
        ],
        out_specs=pl.BlockSpec((BHALF, 2 * EMB), lambda i: (i, 0)),
        out_shape=jax.ShapeDtypeStruct((NPAIR, 2 * EMB), jnp.float32),
    )(tt, tt)


OUTP = 128
BM = 2048


def _mlp_body(x_ref, w1_ref, b1_ref, w2_ref, b2_ref, o_ref):
    h = jnp.dot(x_ref[...], w1_ref[...], preferred_element_type=jnp.float32)
    h = jnp.maximum(h + b1_ref[...], 0.0)
    z = jnp.dot(h, w2_ref[...], preferred_element_type=jnp.float32) + b2_ref[...]
    o_ref[...] = 1.0 / (1.0 + jnp.exp(-z))


def _mlp(x, W1, b1, W2, b2):
    w2p = jnp.zeros((HIDDEN, OUTP), jnp.float32).at[:, :OUT].set(W2)
    b2p = jnp.zeros((1, OUTP), jnp.float32).at[:, :OUT].set(b2)
    out = pl.pallas_call(
        _mlp_body,
        grid=(B // BM,),
        in_specs=[
            pl.BlockSpec((BM, EMB), lambda i: (i, 0)),
            pl.BlockSpec((EMB, HIDDEN), lambda i: (0, 0)),
            pl.BlockSpec((1, HIDDEN), lambda i: (0, 0)),
            pl.BlockSpec((HIDDEN, OUTP), lambda i: (0, 0)),
            pl.BlockSpec((1, OUTP), lambda i: (0, 0)),
        ],
        out_specs=pl.BlockSpec((BM, OUTP), lambda i: (i, 0)),
        out_shape=jax.ShapeDtypeStruct((B, OUTP), jnp.float32),
    )(x, W1, b1.reshape(1, HIDDEN), w2p, b2p)
    return out[:, :OUT]


def kernel(inputs, table, W1, b1, W2, b2):
    table_lin = _repack(table).reshape(2 * NPAIR, EMB)
    idx = inputs.astype(jnp.int32)
    idx = (idx & ~4095) + ((idx & 2047) << 1) + ((idx >> 11) & 1)
    pooled = _pool(idx, table_lin)
    return _mlp(pooled, W1, b1, W2, b2)


# speedup vs baseline: 4.7252x; 1.4817x over previous
import functools

import jax
import jax.numpy as jnp
from jax import lax
from jax.experimental import pallas as pl
from jax.experimental.pallas import tpu as pltpu
from jax.experimental.pallas import tpu_sc as plsc

VOCAB = 1000000
EMB = 64
HIDDEN = 256
OUT = 6
B = 16384
L = 200

LANES = 16
NVREG = EMB // LANES
SEGA = 96
SEGB = L - SEGA
NC, NS = 2, 16
NW = NC * NS
BPW = B // NW
ROWS_H = BPW // 2
HALFV = VOCAB // 2

_mesh = plsc.VectorSubcoreMesh(core_axis_name="c", subcore_axis_name="s")


@functools.partial(
    pl.kernel,
    out_type=jax.ShapeDtypeStruct((B, EMB), jnp.float32),
    mesh=_mesh,
    scratch_types=[
        pltpu.VMEM((ROWS_H, L), jnp.int32),
        pltpu.VMEM((SEGA, EMB), jnp.float32),
        pltpu.VMEM((SEGB, EMB), jnp.float32),
        pltpu.VMEM((SEGA, EMB), jnp.float32),
        pltpu.VMEM((SEGB, EMB), jnp.float32),
        pltpu.VMEM((BPW, EMB), jnp.float32),
        pltpu.SemaphoreType.DMA,
        pltpu.SemaphoreType.DMA,
        pltpu.SemaphoreType.DMA,
        pltpu.SemaphoreType.DMA,
    ],
    compiler_params=pltpu.CompilerParams(use_tc_tiling_on_sc=False),
)
def _pool(idx_hbm, table_hbm, out_hbm, idx_v, a0, a1, b0, b1, out_v,
          sa0, sa1, sb0, sb1):
    wid = lax.axis_index("s") * NC + lax.axis_index("c")
    base = wid * BPW
    bufsets = ((a0, a1), (b0, b1))
    semsets = ((sa0, sa1), (sb0, sb1))
    seglen = (SEGA, SEGB)
    inv_l = 1.0 / L

    def seg(e, b):
        return idx_v.at[e, pl.ds(b * SEGA, seglen[b])]

    def seg_acc(buf, n, accs):
        def jbody(j, a):
            return [a[k] + buf[j, pl.ds(LANES * k, LANES)] for k in range(NVREG)]

        return lax.fori_loop(0, n, jbody, accs, unroll=4)

    def do_row(e, p, h, reissue):
        accs = [jnp.zeros((LANES,), jnp.float32) for _ in range(NVREG)]
        for b in range(2):
            pltpu.make_async_copy(
                table_hbm.at[seg(e, b)], bufsets[p][b], semsets[p][b]
            ).wait()
            accs = seg_acc(bufsets[p][b], seglen[b], accs)
            if reissue:
                pltpu.async_copy(
                    table_hbm.at[seg(e + 2, b)], bufsets[p][b], semsets[p][b]
                )
        r = h * ROWS_H + e
        for k in range(NVREG):
            out_v[r, pl.ds(LANES * k, LANES)] = accs[k] * inv_l

    for h in range(2):
        r0 = base + h * ROWS_H
        pltpu.sync_copy(idx_hbm.at[pl.ds(r0, ROWS_H)], idx_v)
        for p in range(2):
            for b in range(2):
                pltpu.async_copy(
                    table_hbm.at[seg(p, b)], bufsets[p][b], semsets[p][b]
                )

        def pair_body(e2, carry, _h=h):
            for p in range(2):
                do_row(2 * e2 + p, p, _h, True)
            return carry

        lax.fori_loop(0, ROWS_H // 2 - 1, pair_body, 0)

        for p in range(2):
            do_row(ROWS_H - 2 + p, p, h, False)

    pltpu.sync_copy(out_v, out_hbm.at[pl.ds(base, BPW)])


BHALF = 4096
NWIN = pl.cdiv(VOCAB, 2 * BHALF)
NPAIR = NWIN * BHALF
NBLK = pl.cdiv(VOCAB, BHALF)


def _repack_body(x1_ref, x2_ref, o_ref):
    o_ref[...] = jnp.concatenate([x1_ref[...].T, x2_ref[...].T], axis=1)


def _repack(table):
    tt = table.T
    return pl.pallas_call(
        _repack_body,
        grid=(NWIN,),
        in_specs=[
            pl.BlockSpec((EMB, BHALF), lambda i: (0, 2 * i)),
            pl.BlockSpec((EMB, BHALF), lambda i: (0, jnp.minimum(2 * i + 1, NBLK - 1))),
        ],
        out_specs=pl.BlockSpec((BHALF, 2 * EMB), lambda i: (i, 0)),
        out_shape=jax.ShapeDtypeStruct((NPAIR, 2 * EMB), jnp.float32),
    )(tt, tt)


OUTP = 128
BM = 2048


def _mlp_body(x_ref, w1_ref, b1_ref, w2_ref, b2_ref, o_ref):
    h = jnp.dot(x_ref[...], w1_ref[...], preferred_element_type=jnp.float32)
    h = jnp.maximum(h + b1_ref[...], 0.0)
    z = jnp.dot(h, w2_ref[...], preferred_element_type=jnp.float32) + b2_ref[...]
    o_ref[...] = 1.0 / (1.0 + jnp.exp(-z))


def _mlp(x, W1, b1, W2, b2):
    w2p = jnp.zeros((HIDDEN, OUTP), jnp.float32).at[:, :OUT].set(W2)
    b2p = jnp.zeros((1, OUTP), jnp.float32).at[:, :OUT].set(b2)
    out = pl.pallas_call(
        _mlp_body,
        grid=(B // BM,),
        in_specs=[
            pl.BlockSpec((BM, EMB), lambda i: (i, 0)),
            pl.BlockSpec((EMB, HIDDEN), lambda i: (0, 0)),
            pl.BlockSpec((1, HIDDEN), lambda i: (0, 0)),
            pl.BlockSpec((HIDDEN, OUTP), lambda i: (0, 0)),
            pl.BlockSpec((1, OUTP), lambda i: (0, 0)),
        ],
        out_specs=pl.BlockSpec((BM, OUTP), lambda i: (i, 0)),
        out_shape=jax.ShapeDtypeStruct((B, OUTP), jnp.float32),
    )(x, W1, b1.reshape(1, HIDDEN), w2p, b2p)
    return out[:, :OUT]


def kernel(inputs, table, W1, b1, W2, b2):
    table_lin = _repack(table).reshape(2 * NPAIR, EMB)
    idx = inputs.astype(jnp.int32)
    idx = (idx & ~(2 * BHALF - 1)) + ((idx & (BHALF - 1)) << 1) + ((idx // BHALF) & 1)
    pooled = _pool(idx, table_lin)
    return _mlp(pooled, W1, b1, W2, b2)
